# Initial kernel scaffold; baseline (speedup 1.0000x reference)
import jax
import jax.numpy as jnp
from jax.experimental import pallas as pl


def kernel(x, edge_index, edge_weight, traj_seqs, seq_lengths, W, b):
    raise NotImplementedError("write your pallas kernel here")


# speedup vs baseline: 2.8556x; 2.8556x over previous
import jax
import jax.numpy as jnp
from jax.experimental import pallas as pl
from jax.experimental.pallas import tpu as pltpu

N_NODES = 10000
D = 256
BATCH = 16
MAX_LEN = 512


def _mm_body(lens_ref, a_ref, w_ref, b_ref, o_ref):
    i = pl.program_id(0)
    acc = jnp.dot(a_ref[...], w_ref[...], preferred_element_type=jnp.float32)
    acc = acc + b_ref[...]
    acc = jnp.maximum(acc, 0.0)
    L = lens_ref[i]
    mask = jax.lax.broadcasted_iota(jnp.int32, (MAX_LEN, 1), 0) < L
    o_ref[...] = jnp.where(mask, acc, 0.0)


def _final_matmul(a, w, b, lens):
    out = pl.pallas_call(
        _mm_body,
        grid=(BATCH,),
        in_specs=[
            pl.BlockSpec(memory_space=pltpu.SMEM),
            pl.BlockSpec((MAX_LEN, D), lambda i: (i, 0)),
            pl.BlockSpec((D, D), lambda i: (0, 0)),
            pl.BlockSpec((1, D), lambda i: (0, 0)),
        ],
        out_specs=pl.BlockSpec((MAX_LEN, D), lambda i: (i, 0)),
        out_shape=jax.ShapeDtypeStruct((BATCH * MAX_LEN, D), jnp.float32),
    )(lens, a, w, b.reshape(1, D))
    return out.reshape(BATCH, MAX_LEN, D)


def kernel(x, edge_index, edge_weight, traj_seqs, seq_lengths, W, b):
    src = edge_index[0].astype(jnp.int32)
    dst = edge_index[1].astype(jnp.int32)
    lens = seq_lengths.astype(jnp.int32)

    deg = jnp.ones((N_NODES,), jnp.float32).at[dst].add(edge_weight)
    dinv = jax.lax.rsqrt(deg)
    g = dinv[:, None] * x
    acc = g.at[dst].add(edge_weight[:, None] * jnp.take(g, src, axis=0))
    acc = dinv[:, None] * acc
    a = jnp.take(acc, traj_seqs.reshape(-1).astype(jnp.int32), axis=0)
    out = _final_matmul(a, W, b, lens)
    return out, seq_lengths


# speedup vs baseline: 8.9944x; 3.1497x over previous
import functools

import jax
import jax.numpy as jnp
from jax import lax
from jax.experimental import pallas as pl
from jax.experimental.pallas import tpu as pltpu
from jax.experimental.pallas import tpu_sc as plsc

N_NODES = 10000
NPAD = 10240
BAND = NPAD // 16
D = 256
DH = 128
BATCH = 16
MAX_LEN = 512
NT = BATCH * MAX_LEN
TPT = NT // 16
K = 128
TCH = TPT // K
E = 160000
CH = 80
SCH = CH // 8
EPAD = 16 * CH * K

_mesh = plsc.VectorSubcoreMesh(core_axis_name="c", subcore_axis_name="s")


def _scale_rows_by(rows, wbuf, vals16, gi):
    wbuf[pl.ds(16, 16)] = vals16
    for k in range(16):
        bc = plsc.load_gather(wbuf, [jnp.full((16,), 16 + k, jnp.int32)])
        row = gi * 16 + k
        for q in range(DH // 16):
            sl = pl.ds(q * 16, 16)
            rows[row, sl] = rows[row, sl] * bc


@functools.partial(
    pl.kernel,
    out_type=jax.ShapeDtypeStruct((NPAD,), jnp.float32),
    mesh=_mesh,
    compiler_params=pltpu.CompilerParams(needs_layout_passes=False),
    scratch_types=[
        pltpu.VMEM((CH, K), jnp.int32),
        pltpu.VMEM((CH, K), jnp.float32),
        pltpu.VMEM((NPAD,), jnp.float32),
        pltpu.VMEM((320,), jnp.float32),
        pltpu.VMEM((320,), jnp.float32),
        pltpu.VMEM_SHARED((16 * NPAD,), jnp.float32),
    ],
)
def _deg_kernel(dst_r, w_r, deg_out, dstv, wv, degl, acc3, tmp3, shared):
    c = lax.axis_index("c")
    s = lax.axis_index("s")
    pltpu.sync_copy(dst_r.at[s], dstv)
    pltpu.sync_copy(w_r.at[s], wv)
    zero16 = jnp.zeros((16,), jnp.float32)

    def zbody(i, _):
        degl[pl.ds(i * 16, 16)] = zero16
        return 0

    lax.fori_loop(0, NPAD // 16, zbody, 0)

    def ebody(j, _):
        def gbody(gi, _):
            sl = pl.ds(gi * 16, 16)
            plsc.addupdate_scatter(degl, [dstv[j, sl]], wv[j, sl])
            return 0

        lax.fori_loop(0, K // 16, gbody, 0)
        return 0

    lax.fori_loop(0, CH, ebody, 0)
    pltpu.sync_copy(degl, shared.at[pl.ds(s * NPAD, NPAD)])
    plsc.subcore_barrier()

    base = (c * 16 + s) * 320
    one16 = jnp.ones((16,), jnp.float32)

    def ibody(i, _):
        acc3[pl.ds(i * 16, 16)] = one16
        return 0

    lax.fori_loop(0, 20, ibody, 0)

    def rbody(j, _):
        pltpu.sync_copy(shared.at[pl.ds(j * NPAD + base, 320)], tmp3)

        def abody(i, _):
            sl = pl.ds(i * 16, 16)
            acc3[sl] = acc3[sl] + tmp3[sl]
            return 0

        lax.fori_loop(0, 20, abody, 0)
        return 0

    lax.fori_loop(0, 16, rbody, 0)
    pltpu.sync_copy(acc3, deg_out.at[pl.ds(base, 320)])


@functools.partial(
    pl.kernel,
    out_type=jax.ShapeDtypeStruct((2 * NT, DH), jnp.float32),
    mesh=_mesh,
    compiler_params=pltpu.CompilerParams(needs_layout_passes=False),
    scratch_types=[
        pltpu.VMEM((24, K), jnp.int32),
        pltpu.VMEM((K,), jnp.int32),
        pltpu.VMEM((K,), jnp.int32),
        pltpu.VMEM((TCH, K), jnp.int32),
        pltpu.VMEM((NPAD,), jnp.float32),
        pltpu.VMEM((K, DH), jnp.float32),
        pltpu.VMEM((K, DH), jnp.float32),
        pltpu.VMEM((32,), jnp.float32),
        pltpu.SemaphoreType.DMA,
        pltpu.SemaphoreType.DMA,
        pltpu.SemaphoreType.DMA,
        pltpu.SemaphoreType.DMA,
        pltpu.SemaphoreType.DMA,
        pltpu.VMEM_SHARED((NPAD, DH), jnp.float32),
    ],
)
def _gcn_kernel(g2, ed_r, traj_r, dinv_h, gath_out,
                ebuf8, dstix0, dstix1, trajv, dinvv, rows0, rows1, wbuf,
                gsem0, gsem1, ssem0, ssem1, tsem, acc_sh):
    c = lax.axis_index("c")
    s = lax.axis_index("s")
    coff = c * NPAD
    gsems = (gsem0, gsem1)
    ssems = (ssem0, ssem1)
    rowsb = (rows0, rows1)
    dstixb = (dstix0, dstix1)

    pltpu.sync_copy(traj_r.at[s], trajv)
    pltpu.sync_copy(dinv_h, dinvv)

    pltpu.sync_copy(g2.at[pl.ds(coff + s * BAND, BAND)],
                    acc_sh.at[pl.ds(s * BAND, BAND)])
    plsc.subcore_barrier()

    def scale_chunk(b, p):
        def sc_body(gi, _):
            sl = pl.ds(gi * 16, 16)
            dstixb[p][sl] = ebuf8[3 * b + 1, sl]
            w16 = plsc.bitcast(ebuf8[3 * b + 2, sl], jnp.float32)
            _scale_rows_by(rowsb[p], wbuf, w16, gi)
            return 0

        lax.fori_loop(0, K // 16, sc_body, 0)

    def ig(b, p):
        return pltpu.async_copy(g2.at[ebuf8.at[3 * b]], rowsb[p], gsems[p])

    def wg(b, p):
        pltpu.make_async_copy(g2.at[ebuf8.at[3 * b]], rowsb[p],
                              gsems[p]).wait()

    def isc(p):
        return pltpu.async_copy(rowsb[p], acc_sh.at[dstixb[p]],
                                ssems[p], add=True)

    pltpu.sync_copy(ed_r.at[c, s, 0], ebuf8)
    ig(0, 0)

    def super_body(k, _):
        gh = [None, None]
        sh = [None, None]
        for b in range(8):
            p = b % 2
            q = 1 - p
            if gh[p] is None:
                wg(b, p)
            else:
                gh[p].wait()
            if sh[q] is not None:
                sh[q].wait()
            if b < 7:
                gh[q] = ig(b + 1, q)
                scale_chunk(b, p)
                sh[p] = isc(p)
            else:
                scale_chunk(b, p)
                last = isc(p)

                @pl.when(k + 1 < SCH)
                def _():
                    pltpu.sync_copy(ed_r.at[c, s, k + 1], ebuf8)
                    ig(0, q)

                last.wait()
        return 0

    lax.fori_loop(0, SCH, super_body, 0)
    plsc.subcore_barrier()

    def tbody(t, _):
        pltpu.async_copy(acc_sh.at[trajv.at[t]], rows0, tsem).wait()

        def sbody(gi, _):
            d16 = plsc.load_gather(dinvv, [trajv[t, pl.ds(gi * 16, 16)]])
            _scale_rows_by(rows0, wbuf, d16, gi)
            return 0

        lax.fori_loop(0, K // 16, sbody, 0)
        pltpu.sync_copy(rows0,
                        gath_out.at[pl.ds(c * NT + s * TPT + t * K, K)])
        return 0

    lax.fori_loop(0, TCH, tbody, 0)


def _mm_body(lens_ref, a0_ref, a1_ref, w0_ref, w1_ref, b_ref, o_ref):
    i = pl.program_id(0)
    acc = jnp.dot(a0_ref[...], w0_ref[...], preferred_element_type=jnp.float32)
    acc = acc + jnp.dot(a1_ref[...], w1_ref[...],
                        preferred_element_type=jnp.float32)
    acc = jnp.maximum(acc + b_ref[...], 0.0)
    L = lens_ref[i]
    mask = jax.lax.broadcasted_iota(jnp.int32, (MAX_LEN, 1), 0) < L
    o_ref[...] = jnp.where(mask, acc, 0.0)


def _final_matmul(a0, a1, w, b, lens):
    out = pl.pallas_call(
        _mm_body,
        grid=(BATCH,),
        in_specs=[
            pl.BlockSpec(memory_space=pltpu.SMEM),
            pl.BlockSpec((MAX_LEN, DH), lambda i: (i, 0)),
            pl.BlockSpec((MAX_LEN, DH), lambda i: (i, 0)),
            pl.BlockSpec((DH, D), lambda i: (0, 0)),
            pl.BlockSpec((DH, D), lambda i: (0, 0)),
            pl.BlockSpec((1, D), lambda i: (0, 0)),
        ],
        out_specs=pl.BlockSpec((MAX_LEN, D), lambda i: (i, 0)),
        out_shape=jax.ShapeDtypeStruct((NT, D), jnp.float32),
    )(lens, a0, a1, w[:DH], w[DH:], b.reshape(1, D))
    return out.reshape(BATCH, MAX_LEN, D)


def kernel(x, edge_index, edge_weight, traj_seqs, seq_lengths, W, b):
    src = edge_index[0].astype(jnp.int32)
    dst = edge_index[1].astype(jnp.int32)
    w = edge_weight.astype(jnp.float32)
    lens = seq_lengths.astype(jnp.int32)

    pad = EPAD - E
    srcp = jnp.concatenate([src, jnp.zeros((pad,), jnp.int32)]).reshape(16, CH, K)
    dstp = jnp.concatenate([dst, jnp.zeros((pad,), jnp.int32)]).reshape(16, CH, K)
    wp = jnp.concatenate([w, jnp.zeros((pad,), jnp.float32)]).reshape(16, CH, K)
    trajr = traj_seqs.astype(jnp.int32).reshape(16, TCH, K)

    deg = _deg_kernel(dstp, wp)
    dinv = jax.lax.rsqrt(deg)
    xpad = jnp.zeros((NPAD, D), jnp.float32).at[:N_NODES].set(x)
    g = dinv[:, None] * xpad
    g2 = jnp.concatenate([g[:, :DH], g[:, DH:]], axis=0)

    srcp4 = srcp.reshape(16, SCH, 8, K)
    dstp4 = dstp.reshape(16, SCH, 8, K)
    wbits4 = jax.lax.bitcast_convert_type(wp, jnp.int32).reshape(16, SCH, 8, K)
    ed = jnp.stack(
        [jnp.stack([srcp4, dstp4, wbits4], axis=3),
         jnp.stack([srcp4 + NPAD, dstp4, wbits4], axis=3)],
        axis=0).reshape(2, 16, SCH, 24, K)

    gath = _gcn_kernel(g2, ed, trajr, dinv)
    out = _final_matmul(gath[:NT], gath[NT:], W, b, lens)
    return out, seq_lengths
